# baseline (speedup 1.0000x reference)
import functools

import jax
import jax.numpy as jnp
from jax import lax
from jax.experimental import pallas as pl
from jax.experimental.pallas import tpu as pltpu
from jax.experimental.pallas import tpu_sc as plsc

B = 4096
D = 768
T = 8
C = 512
L = 12

BLK = 256
BP = B + T * BLK
NB = BP // BLK
GRP = 8
BH_LEN = 32

NC = 2
NS = 16
NW = NC * NS
CHUNK = B // NW
NCH = 4
SUB = CHUNK // NCH


def _route_body(votes_ref, pos_ref, bh_ref):
    v = votes_ref[...]
    cnt = jnp.concatenate(
        [jnp.sum((v == t).astype(jnp.int32), axis=0, keepdims=True)
         for t in range(T)], axis=0)
    best = cnt[T - 1:T]
    m = jnp.full((1, B), T - 1, jnp.int32)
    for t in range(T - 2, -1, -1):
        c = cnt[t:t + 1]
        take = c >= best
        best = jnp.where(take, c, best)
        m = jnp.where(take, t, m)
    oh16 = (lax.broadcasted_iota(jnp.int32, (T, B), 0) == m).astype(jnp.int16)
    cs16 = oh16
    d = 1
    while d < B:
        shifted = jnp.concatenate(
            [jnp.zeros((T, d), jnp.int16), cs16[:, :B - d]], axis=1)
        cs16 = cs16 + shifted
        d *= 2
    oh = oh16.astype(jnp.int32)
    cs = cs16.astype(jnp.int32)
    total = cs[:, B - 1:B]
    pc = ((total + BLK - 1) // BLK) * BLK
    offs = []
    acc = jnp.zeros((1, 1), jnp.int32)
    for t in range(T):
        offs.append(acc)
        acc = acc + pc[t:t + 1]
    off = jnp.concatenate(offs, axis=0)
    rank = jnp.sum(oh * cs, axis=0, keepdims=True) - 1
    base = jnp.sum(oh * off, axis=0, keepdims=True)
    pos_ref[...] = base + rank
    nbase = lax.broadcasted_iota(jnp.int32, (1, BH_LEN), 1) * BLK
    s = jnp.zeros((1, BH_LEN), jnp.int32)
    for t in range(T):
        s = s + (nbase >= off[t, 0]).astype(jnp.int32)
    thead = s - 1
    used2 = (acc + GRP * BLK - 1) // (GRP * BLK)
    nbs = lax.broadcasted_iota(jnp.int32, (1, BH_LEN), 1)
    fmap = jnp.minimum(nbs, used2 - 1)
    omap = jnp.minimum(nbs, used2)
    bh_ref[...] = jnp.concatenate([thead, fmap, omap, thead], axis=0)


def _mm_body(tbl_ref, f_ref, w_ref, b_ref, o_ref, w16_ref):
    j = pl.program_id(0)

    @pl.when(j == 0)
    def _cast_w():
        w16_ref[...] = w_ref[...].astype(jnp.bfloat16)

    @pl.when(tbl_ref[1, j] == j)
    def _compute():
        f = f_ref[...].astype(jnp.bfloat16)
        for k in range(GRP):
            t = tbl_ref[0, GRP * j + k]
            o_ref[k * BLK:(k + 1) * BLK] = (
                jnp.dot(f[k * BLK:(k + 1) * BLK], w16_ref[t],
                        preferred_element_type=jnp.float32) + b_ref[t])


def _scatter_feats_body(features, pos2d, sorted_f, i0, i1, i2, i3, rows_v,
                        semi, semr, sems):
    wid = lax.axis_index("s") * NC + lax.axis_index("c")
    bs = wid * CHUNK
    idx = [i0, i1, i2, i3]
    cps = [
        pltpu.async_copy(pos2d.at[0, pl.ds(bs + h * SUB, SUB)], idx[h], semi)
        for h in range(NCH)
    ]
    loads = [
        pltpu.async_copy(features.at[pl.ds(bs + h * SUB, SUB)],
                         rows_v.at[pl.ds(h * SUB, SUB)], semr)
        for h in range(NCH)
    ]
    stores = []
    for h in range(NCH):
        cps[h].wait()
        loads[h].wait()
        stores.append(
            pltpu.async_copy(rows_v.at[pl.ds(h * SUB, SUB)],
                             sorted_f.at[idx[h]], sems))
    for c in stores:
        c.wait()


def _gather_out_body(sorted_out, pos2d, out, i0, i1, i2, i3, rows_v, semi,
                     semr, sems):
    wid = lax.axis_index("s") * NC + lax.axis_index("c")
    bs = wid * CHUNK
    idx = [i0, i1, i2, i3]
    cps = [
        pltpu.async_copy(pos2d.at[0, pl.ds(bs + h * SUB, SUB)], idx[h], semi)
        for h in range(NCH)
    ]
    loads = []
    for h in range(NCH):
        cps[h].wait()
        loads.append(
            pltpu.async_copy(sorted_out.at[idx[h]],
                             rows_v.at[pl.ds(h * SUB, SUB)], semr))
    stores = []
    for h in range(NCH):
        loads[h].wait()
        stores.append(
            pltpu.async_copy(rows_v.at[pl.ds(h * SUB, SUB)],
                             out.at[pl.ds(bs + h * SUB, SUB)], sems))
    for c in stores:
        c.wait()


def kernel(features, votes, heads_w, heads_b):
    votes = votes.astype(jnp.int32)

    pos2d, bh2d = pl.pallas_call(
        _route_body,
        out_shape=[
            jax.ShapeDtypeStruct((1, B), jnp.int32),
            jax.ShapeDtypeStruct((4, BH_LEN), jnp.int32),
        ],
    )(votes)

    mesh = plsc.VectorSubcoreMesh(core_axis_name="c", subcore_axis_name="s")

    scatter_feats = functools.partial(
        pl.kernel,
        mesh=mesh,
        out_type=jax.ShapeDtypeStruct((BP, D), jnp.float32),
        scratch_types=[
            pltpu.VMEM((SUB,), jnp.int32),
            pltpu.VMEM((SUB,), jnp.int32),
            pltpu.VMEM((SUB,), jnp.int32),
            pltpu.VMEM((SUB,), jnp.int32),
            pltpu.VMEM((CHUNK, D), jnp.float32),
            pltpu.SemaphoreType.DMA,
            pltpu.SemaphoreType.DMA,
            pltpu.SemaphoreType.DMA,
        ],
    )(_scatter_feats_body)
    sorted_f = scatter_feats(features, pos2d)

    sorted_out = pl.pallas_call(
        _mm_body,
        grid_spec=pltpu.PrefetchScalarGridSpec(
            num_scalar_prefetch=1,
            grid=(NB // GRP,),
            in_specs=[
                pl.BlockSpec((GRP * BLK, D), lambda j, s: (s[1, j], 0)),
                pl.BlockSpec((T, D, C), lambda j, s: (0, 0, 0)),
                pl.BlockSpec((T, 1, C), lambda j, s: (0, 0, 0)),
            ],
            out_specs=pl.BlockSpec((GRP * BLK, C), lambda j, s: (s[2, j], 0)),
            scratch_shapes=[pltpu.VMEM((T, D, C), jnp.bfloat16)],
        ),
        out_shape=jax.ShapeDtypeStruct((BP + GRP * BLK, C), jnp.float32),
        compiler_params=pltpu.CompilerParams(
            dimension_semantics=("arbitrary",)),
    )(bh2d, sorted_f, heads_w, heads_b.reshape(T, 1, C))

    gather_out = functools.partial(
        pl.kernel,
        mesh=mesh,
        out_type=jax.ShapeDtypeStruct((B, C), jnp.float32),
        scratch_types=[
            pltpu.VMEM((SUB,), jnp.int32),
            pltpu.VMEM((SUB,), jnp.int32),
            pltpu.VMEM((SUB,), jnp.int32),
            pltpu.VMEM((SUB,), jnp.int32),
            pltpu.VMEM((CHUNK, C), jnp.float32),
            pltpu.SemaphoreType.DMA,
            pltpu.SemaphoreType.DMA,
            pltpu.SemaphoreType.DMA,
        ],
    )(_gather_out_body)
    return gather_out(sorted_out, pos2d)


# speedup vs baseline: 1.0205x; 1.0205x over previous
import functools

import jax
import jax.numpy as jnp
from jax import lax
from jax.experimental import pallas as pl
from jax.experimental.pallas import tpu as pltpu
from jax.experimental.pallas import tpu_sc as plsc

B = 4096
D = 768
T = 8
C = 512
L = 12

BLK = 256
BP = B + T * BLK
NB = BP // BLK
GRP = 4
BH_LEN = 32

NC = 2
NS = 16
NW = NC * NS
CHUNK = B // NW
NCH = 4
SUB = CHUNK // NCH


def _route_body(votes_ref, pos_ref, bh_ref):
    v = votes_ref[...]
    cnt = jnp.concatenate(
        [jnp.sum((v == t).astype(jnp.int32), axis=0, keepdims=True)
         for t in range(T)], axis=0)
    best = cnt[T - 1:T]
    m = jnp.full((1, B), T - 1, jnp.int32)
    for t in range(T - 2, -1, -1):
        c = cnt[t:t + 1]
        take = c >= best
        best = jnp.where(take, c, best)
        m = jnp.where(take, t, m)
    oh16 = (lax.broadcasted_iota(jnp.int32, (T, B), 0) == m).astype(jnp.int16)
    cs16 = oh16
    d = 1
    while d < B:
        shifted = jnp.concatenate(
            [jnp.zeros((T, d), jnp.int16), cs16[:, :B - d]], axis=1)
        cs16 = cs16 + shifted
        d *= 2
    oh = oh16.astype(jnp.int32)
    cs = cs16.astype(jnp.int32)
    total = cs[:, B - 1:B]
    pc = ((total + BLK - 1) // BLK) * BLK
    offs = []
    acc = jnp.zeros((1, 1), jnp.int32)
    for t in range(T):
        offs.append(acc)
        acc = acc + pc[t:t + 1]
    off = jnp.concatenate(offs, axis=0)
    rank = jnp.sum(oh * cs, axis=0, keepdims=True) - 1
    base = jnp.sum(oh * off, axis=0, keepdims=True)
    pos_ref[...] = base + rank
    nbase = lax.broadcasted_iota(jnp.int32, (1, BH_LEN), 1) * BLK
    s = jnp.zeros((1, BH_LEN), jnp.int32)
    for t in range(T):
        s = s + (nbase >= off[t, 0]).astype(jnp.int32)
    thead = s - 1
    used2 = (acc + GRP * BLK - 1) // (GRP * BLK)
    nbs = lax.broadcasted_iota(jnp.int32, (1, BH_LEN), 1)
    fmap = jnp.minimum(nbs, used2 - 1)
    omap = jnp.minimum(nbs, used2)
    bh_ref[...] = jnp.concatenate([thead, fmap, omap, thead], axis=0)


def _mm_body(tbl_ref, f_ref, w_ref, b_ref, o_ref, w16_ref):
    j = pl.program_id(0)

    @pl.when(j == 0)
    def _cast_w():
        w16_ref[...] = w_ref[...].astype(jnp.bfloat16)

    @pl.when(tbl_ref[1, j] == j)
    def _compute():
        f = f_ref[...].astype(jnp.bfloat16)
        for k in range(GRP):
            t = tbl_ref[0, GRP * j + k]
            o_ref[k * BLK:(k + 1) * BLK] = (
                jnp.dot(f[k * BLK:(k + 1) * BLK], w16_ref[t],
                        preferred_element_type=jnp.float32) + b_ref[t])


def _scatter_feats_body(features, pos2d, sorted_f, i0, i1, i2, i3, rows_v,
                        semi, semr, sems):
    wid = lax.axis_index("s") * NC + lax.axis_index("c")
    bs = wid * CHUNK
    idx = [i0, i1, i2, i3]
    cps = [
        pltpu.async_copy(pos2d.at[0, pl.ds(bs + h * SUB, SUB)], idx[h], semi)
        for h in range(NCH)
    ]
    loads = [
        pltpu.async_copy(features.at[pl.ds(bs + h * SUB, SUB)],
                         rows_v.at[pl.ds(h * SUB, SUB)], semr)
        for h in range(NCH)
    ]
    stores = []
    for h in range(NCH):
        cps[h].wait()
        loads[h].wait()
        stores.append(
            pltpu.async_copy(rows_v.at[pl.ds(h * SUB, SUB)],
                             sorted_f.at[idx[h]], sems))
    for c in stores:
        c.wait()


def _gather_out_body(sorted_out, pos2d, out, i0, i1, i2, i3, rows_v, semi,
                     semr, sems):
    wid = lax.axis_index("s") * NC + lax.axis_index("c")
    bs = wid * CHUNK
    idx = [i0, i1, i2, i3]
    cps = [
        pltpu.async_copy(pos2d.at[0, pl.ds(bs + h * SUB, SUB)], idx[h], semi)
        for h in range(NCH)
    ]
    loads = []
    for h in range(NCH):
        cps[h].wait()
        loads.append(
            pltpu.async_copy(sorted_out.at[idx[h]],
                             rows_v.at[pl.ds(h * SUB, SUB)], semr))
    stores = []
    for h in range(NCH):
        loads[h].wait()
        stores.append(
            pltpu.async_copy(rows_v.at[pl.ds(h * SUB, SUB)],
                             out.at[pl.ds(bs + h * SUB, SUB)], sems))
    for c in stores:
        c.wait()


def kernel(features, votes, heads_w, heads_b):
    votes = votes.astype(jnp.int32)

    pos2d, bh2d = pl.pallas_call(
        _route_body,
        out_shape=[
            jax.ShapeDtypeStruct((1, B), jnp.int32),
            jax.ShapeDtypeStruct((4, BH_LEN), jnp.int32),
        ],
    )(votes)

    mesh = plsc.VectorSubcoreMesh(core_axis_name="c", subcore_axis_name="s")

    scatter_feats = functools.partial(
        pl.kernel,
        mesh=mesh,
        out_type=jax.ShapeDtypeStruct((BP, D), jnp.float32),
        scratch_types=[
            pltpu.VMEM((SUB,), jnp.int32),
            pltpu.VMEM((SUB,), jnp.int32),
            pltpu.VMEM((SUB,), jnp.int32),
            pltpu.VMEM((SUB,), jnp.int32),
            pltpu.VMEM((CHUNK, D), jnp.float32),
            pltpu.SemaphoreType.DMA,
            pltpu.SemaphoreType.DMA,
            pltpu.SemaphoreType.DMA,
        ],
    )(_scatter_feats_body)
    sorted_f = scatter_feats(features, pos2d)

    sorted_out = pl.pallas_call(
        _mm_body,
        grid_spec=pltpu.PrefetchScalarGridSpec(
            num_scalar_prefetch=1,
            grid=(NB // GRP,),
            in_specs=[
                pl.BlockSpec((GRP * BLK, D), lambda j, s: (s[1, j], 0)),
                pl.BlockSpec((T, D, C), lambda j, s: (0, 0, 0)),
                pl.BlockSpec((T, 1, C), lambda j, s: (0, 0, 0)),
            ],
            out_specs=pl.BlockSpec((GRP * BLK, C), lambda j, s: (s[2, j], 0)),
            scratch_shapes=[pltpu.VMEM((T, D, C), jnp.bfloat16)],
        ),
        out_shape=jax.ShapeDtypeStruct((BP + GRP * BLK, C), jnp.float32),
        compiler_params=pltpu.CompilerParams(
            dimension_semantics=("arbitrary",)),
    )(bh2d, sorted_f, heads_w, heads_b.reshape(T, 1, C))

    gather_out = functools.partial(
        pl.kernel,
        mesh=mesh,
        out_type=jax.ShapeDtypeStruct((B, C), jnp.float32),
        scratch_types=[
            pltpu.VMEM((SUB,), jnp.int32),
            pltpu.VMEM((SUB,), jnp.int32),
            pltpu.VMEM((SUB,), jnp.int32),
            pltpu.VMEM((SUB,), jnp.int32),
            pltpu.VMEM((CHUNK, C), jnp.float32),
            pltpu.SemaphoreType.DMA,
            pltpu.SemaphoreType.DMA,
            pltpu.SemaphoreType.DMA,
        ],
    )(_gather_out_body)
    return gather_out(sorted_out, pos2d)
